# baseline (speedup 1.0000x reference)
import functools

import jax
import jax.numpy as jnp
from jax import lax
from jax.experimental import pallas as pl
from jax.experimental.pallas import tpu as pltpu
from jax.experimental.pallas import tpu_sc as plsc

VOCAB = 1000000
EMB_DIM = 32
BATCH = 16384

_PACK = 128 // EMB_DIM
_ROWS = VOCAB // _PACK
_NC = 2
_NS = 16
_NW = _NC * _NS
_B_PER_W = BATCH // _NW
_CHUNK = 128
_NCHUNK = _B_PER_W // _CHUNK
_NBLK = VOCAB // 128 + 1

_mesh = plsc.VectorSubcoreMesh(core_axis_name="c", subcore_axis_name="s")
_params = pltpu.CompilerParams(needs_layout_passes=False)


@functools.partial(
    pl.kernel,
    mesh=_mesh,
    out_type=jax.ShapeDtypeStruct((_ROWS, 128), jnp.float32),
    scratch_types=[
        pltpu.VMEM((EMB_DIM, 128), jnp.float32),
        pltpu.VMEM((32, 128), jnp.float32),
        pltpu.VMEM((64, EMB_DIM), jnp.float32),
        pltpu.SemaphoreType.DMA,
    ],
    compiler_params=_params,
)
def _repack(table_hbm, tail_hbm, t128_hbm, blk_v, out_v, tail_v, sem):
    wid = lax.axis_index("s") * _NC + lax.axis_index("c")

    def transpose_block(nm):
        def mbody(m, _):
            for g in range(8):
                rows = jnp.arange(16, dtype=jnp.int32) + 16 * (g % 2)
                cols = jnp.full((16,), 4 * m + (g // 2), jnp.int32)
                out_v[m, pl.ds(g * 16, 16)] = plsc.load_gather(
                    blk_v, [rows, cols]
                )
            return ()

        lax.fori_loop(0, nm, mbody, (), unroll=False)

    def body(jj, _):
        j = jj * _NW + wid

        @pl.when(j < _NBLK - 1)
        def _():
            src = pl.multiple_of(j * 128, 128)
            pltpu.sync_copy(table_hbm.at[:, pl.ds(src, 128)], blk_v)
            transpose_block(32)
            dst = pl.multiple_of(j * 32, 32)
            pltpu.sync_copy(out_v, t128_hbm.at[pl.ds(dst, 32)])

        @pl.when(j == _NBLK - 1)
        def _():
            pltpu.sync_copy(tail_hbm, tail_v)

            def mbody(m, _):
                for g in range(8):
                    rows = jnp.full((16,), 4 * m + (g // 2), jnp.int32)
                    cols = jnp.arange(16, dtype=jnp.int32) + 16 * (g % 2)
                    out_v[m, pl.ds(g * 16, 16)] = plsc.load_gather(
                        tail_v, [rows, cols]
                    )
                return ()

            lax.fori_loop(0, 16, mbody, (), unroll=False)
            pltpu.sync_copy(
                out_v.at[pl.ds(0, 16)],
                t128_hbm.at[pl.ds(_ROWS - 16, 16)],
            )

        return ()

    lax.fori_loop(0, _NBLK // _NW + 1, body, (), unroll=False)


@functools.partial(
    pl.kernel,
    mesh=_mesh,
    out_type=jax.ShapeDtypeStruct((BATCH, EMB_DIM), jnp.float32),
    scratch_types=[
        pltpu.VMEM((_NCHUNK, _CHUNK), jnp.int32),
        pltpu.VMEM((_B_PER_W,), jnp.int32),
        pltpu.VMEM((2, _CHUNK, 128), jnp.float32),
        pltpu.VMEM((_B_PER_W, EMB_DIM), jnp.float32),
        pltpu.SemaphoreType.DMA,
    ],
    compiler_params=_params,
)
def _lookup(q_hbm, roff_hbm, t128_hbm, out_hbm, q_v, roff_v, rows_v, out_v,
            sem):
    wid = lax.axis_index("s") * _NC + lax.axis_index("c")
    pltpu.sync_copy(q_hbm.at[wid], q_v)
    pltpu.sync_copy(roff_hbm.at[wid], roff_v)

    def start(j):
        return pltpu.async_copy(
            t128_hbm.at[q_v.at[j]], rows_v.at[j % 2], sem
        )

    pending = start(0)
    for j in range(_NCHUNK):
        nxt = start(j + 1) if j + 1 < _NCHUNK else None
        pending.wait()
        pending = nxt
        buf = rows_v.at[j % 2]

        def body(g, _, j=j, buf=buf):
            i0 = j * _CHUNK + g * 16
            rows = jnp.arange(16, dtype=jnp.int32) + (g * 16)
            orows = rows + (j * _CHUNK)
            cols0 = roff_v[pl.ds(i0, 16)]
            for d in range(EMB_DIM):
                vals = plsc.load_gather(buf, [rows, cols0 + d])
                dcol = jnp.full((16,), d, dtype=jnp.int32)
                plsc.store_scatter(out_v, [orows, dcol], vals)
            return ()

        lax.fori_loop(0, _CHUNK // 16, body, (), unroll=False)

    pltpu.sync_copy(out_v, out_hbm.at[pl.ds(wid * _B_PER_W, _B_PER_W)])


def kernel(input_x_pos, Emb):
    idx = input_x_pos.astype(jnp.int32)
    q = (idx // _PACK).reshape(_NW, _NCHUNK, _CHUNK)
    roff = ((idx % _PACK) * EMB_DIM).reshape(_NW, _B_PER_W)
    tail = jax.lax.slice(Emb, (VOCAB - 64, 0), (VOCAB, EMB_DIM))
    t128 = _repack(Emb.T, tail)
    return _lookup(q, roff, t128)


# speedup vs baseline: 1.3222x; 1.3222x over previous
import functools

import jax
import jax.numpy as jnp
from jax import lax
from jax.experimental import pallas as pl
from jax.experimental.pallas import tpu as pltpu
from jax.experimental.pallas import tpu_sc as plsc

VOCAB = 1000000
EMB_DIM = 32
BATCH = 16384

_PACK = 128 // EMB_DIM
_ROWS = VOCAB // _PACK
_NC = 2
_NS = 16
_NW = _NC * _NS
_B_PER_W = BATCH // _NW
_CHUNK = 128
_NCHUNK = _B_PER_W // _CHUNK
_NBLK = VOCAB // 128 + 1

_mesh = plsc.VectorSubcoreMesh(core_axis_name="c", subcore_axis_name="s")
_params = pltpu.CompilerParams(needs_layout_passes=False)


@functools.partial(
    pl.kernel,
    mesh=_mesh,
    out_type=jax.ShapeDtypeStruct((_ROWS, 128), jnp.float32),
    scratch_types=[
        pltpu.VMEM((2, EMB_DIM, 128), jnp.float32),
        pltpu.VMEM((2, 32, 128), jnp.float32),
        pltpu.VMEM((64, EMB_DIM), jnp.float32),
        pltpu.SemaphoreType.DMA,
        pltpu.SemaphoreType.DMA,
    ],
    compiler_params=_params,
)
def _repack(table_hbm, tail_hbm, t128_hbm, blk_v, out_v, tail_v, sem_in,
            sem_out):
    wid = lax.axis_index("s") * _NC + lax.axis_index("c")
    nfull = _NBLK - 1
    nb = (nfull - wid + _NW - 1) // _NW

    def start_in(jj, p):
        src = pl.multiple_of((jj * _NW + wid) * 128, 128)
        return pltpu.async_copy(
            table_hbm.at[:, pl.ds(src, 128)], blk_v.at[p], sem_in
        )

    start_in(0, 0)

    def body(jj, _):
        p = jj % 2
        pltpu.make_async_copy(
            table_hbm.at[:, pl.ds(0, 128)], blk_v.at[p], sem_in
        ).wait()

        @pl.when(jj + 1 < nb)
        def _():
            start_in(jj + 1, 1 - p)

        @pl.when(jj >= 2)
        def _():
            pltpu.make_async_copy(
                out_v.at[p], t128_hbm.at[pl.ds(0, 32)], sem_out
            ).wait()

        buf = blk_v.at[p]
        obuf = out_v.at[p]
        for m in range(32):
            for g in range(8):
                rows = jnp.arange(16, dtype=jnp.int32) + 16 * (g % 2)
                cols = jnp.full((16,), 4 * m + (g // 2), jnp.int32)
                obuf[m, pl.ds(g * 16, 16)] = plsc.load_gather(
                    buf, [rows, cols]
                )

        dst = pl.multiple_of((jj * _NW + wid) * 32, 32)
        pltpu.async_copy(obuf, t128_hbm.at[pl.ds(dst, 32)], sem_out)
        return ()

    lax.fori_loop(0, nb, body, (), unroll=False)

    for p in range(2):
        pltpu.make_async_copy(
            out_v.at[p], t128_hbm.at[pl.ds(0, 32)], sem_out
        ).wait()

    @pl.when(wid == 0)
    def _():
        pltpu.sync_copy(tail_hbm, tail_v)
        obuf = out_v.at[0]
        for m in range(16):
            for g in range(8):
                rows = jnp.full((16,), 4 * m + (g // 2), jnp.int32)
                cols = jnp.arange(16, dtype=jnp.int32) + 16 * (g % 2)
                obuf[m, pl.ds(g * 16, 16)] = plsc.load_gather(
                    tail_v, [rows, cols]
                )
        pltpu.sync_copy(
            obuf.at[pl.ds(0, 16)], t128_hbm.at[pl.ds(_ROWS - 16, 16)]
        )


@functools.partial(
    pl.kernel,
    mesh=_mesh,
    out_type=jax.ShapeDtypeStruct((BATCH, EMB_DIM), jnp.float32),
    scratch_types=[
        pltpu.VMEM((_NCHUNK, _CHUNK), jnp.int32),
        pltpu.VMEM((_B_PER_W,), jnp.int32),
        pltpu.VMEM((2, _CHUNK, 128), jnp.float32),
        pltpu.VMEM((_B_PER_W, EMB_DIM), jnp.float32),
        pltpu.SemaphoreType.DMA,
    ],
    compiler_params=_params,
)
def _lookup(q_hbm, roff_hbm, t128_hbm, out_hbm, q_v, roff_v, rows_v, out_v,
            sem):
    wid = lax.axis_index("s") * _NC + lax.axis_index("c")
    pltpu.sync_copy(q_hbm.at[wid], q_v)
    pltpu.sync_copy(roff_hbm.at[wid], roff_v)

    def start(j):
        return pltpu.async_copy(
            t128_hbm.at[q_v.at[j]], rows_v.at[j % 2], sem
        )

    pending = start(0)
    for j in range(_NCHUNK):
        nxt = start(j + 1) if j + 1 < _NCHUNK else None
        pending.wait()
        pending = nxt
        buf = rows_v.at[j % 2]

        def body(g, _, j=j, buf=buf):
            i0 = j * _CHUNK + g * 16
            rows = jnp.arange(16, dtype=jnp.int32) + (g * 16)
            orows = rows + (j * _CHUNK)
            cols0 = roff_v[pl.ds(i0, 16)]
            for d in range(EMB_DIM):
                vals = plsc.load_gather(buf, [rows, cols0 + d])
                dcol = jnp.full((16,), d, dtype=jnp.int32)
                plsc.store_scatter(out_v, [orows, dcol], vals)
            return ()

        lax.fori_loop(0, _CHUNK // 16, body, (), unroll=False)

    pltpu.sync_copy(out_v, out_hbm.at[pl.ds(wid * _B_PER_W, _B_PER_W)])


def kernel(input_x_pos, Emb):
    idx = input_x_pos.astype(jnp.int32)
    q = (idx // _PACK).reshape(_NW, _NCHUNK, _CHUNK)
    roff = ((idx % _PACK) * EMB_DIM).reshape(_NW, _B_PER_W)
    tail = jax.lax.slice(Emb, (VOCAB - 64, 0), (VOCAB, EMB_DIM))
    t128 = _repack(Emb.T, tail)
    return _lookup(q, roff, t128)
